# speedup vs baseline: 1.9342x; 1.9342x over previous
import functools

import jax
import jax.numpy as jnp
import numpy as np
from jax.experimental import pallas as pl

B, P, C = 4096, 2000, 3

KB = (np.uint32(0x6D3E048F), np.uint32(0x1022172D))
KC = (np.uint32(0x03D7B32D), np.uint32(0xADD083F4))
KE = (np.uint32(0x92FB20EA), np.uint32(0x0F38D913))

TINY = np.float32(np.finfo(np.float32).tiny)

ROWS = 256


def _gumbel_bits(idx_u32, key):
    ks0 = jnp.uint32(key[0])
    ks1 = jnp.uint32(key[1])
    ks2 = ks0 ^ ks1 ^ jnp.uint32(0x1BD11BDA)
    ks = (ks0, ks1, ks2)
    x0 = jnp.full_like(idx_u32, ks0)
    x1 = idx_u32 + ks1
    rotations = ((13, 15, 26, 6), (17, 29, 16, 24))
    for i in range(5):
        for r in rotations[i % 2]:
            x0 = x0 + x1
            x1 = ((x1 << np.uint32(r)) | (x1 >> np.uint32(32 - r))) ^ x0
        x0 = x0 + ks[(i + 1) % 3]
        x1 = x1 + ks[(i + 2) % 3] + jnp.uint32(i + 1)
    bits = x0 ^ x1
    fb = (bits >> jnp.uint32(9)) | jnp.uint32(0x3F800000)
    f = jax.lax.bitcast_convert_type(fb, jnp.float32) - jnp.float32(1.0)
    u = jnp.maximum(f * jnp.float32(1.0) + TINY, TINY)
    return -jnp.log(-jnp.log(u))


def _first_argmax(s, col):
    m = jnp.max(s, axis=1, keepdims=True)
    return jnp.min(jnp.where(s == m, col, P), axis=1, keepdims=True)


def _gather_row(x, col, idx):
    return jnp.sum(jnp.where(col == idx, x, jnp.zeros_like(x)), axis=1,
                   keepdims=True)


def _agent_kernel(tour_ref, pv_ref, bt_ref, cp_ref, ep_ref,
                  btc_ref, cc_ref, etic_ref, alp_ref, vc_ref):
    i = pl.program_id(0)
    bt = bt_ref[...]
    cp = cp_ref[...]
    ep = ep_ref[...]
    tour = tour_ref[...]

    row = jax.lax.broadcasted_iota(jnp.int32, (ROWS, P), 0) + i * ROWS
    col = jax.lax.broadcasted_iota(jnp.int32, (ROWS, P), 1)

    bt_max = jnp.max(bt, axis=1, keepdims=True)
    bt_lse = jnp.log(jnp.sum(jnp.exp(bt - bt_max), axis=1, keepdims=True))

    cl = jnp.log(jnp.maximum(cp, jnp.float32(1e-30)))
    cp_max = jnp.max(cp, axis=1, keepdims=True)
    cp_lse = jnp.log(jnp.sum(jnp.exp(cp - cp_max), axis=1, keepdims=True))

    btc_cols, cc_cols, alp_cols, e_cols = [], [], [], []
    for c in range(C):
        gb = _gumbel_bits(((c * B + row) * P + col).astype(jnp.uint32), KB)
        b_idx = _first_argmax(bt + gb, col)
        b_alp = _gather_row(bt, col, b_idx) - bt_max - bt_lse

        gc = _gumbel_bits(((c * B + row) * P + col).astype(jnp.uint32), KC)
        c_idx = _first_argmax(cl + gc, col)
        c_alp = _gather_row(cp, col, c_idx) - cp_max - cp_lse

        masked = jnp.where(col == c_idx, jnp.float32(1e-9), ep)
        s_sum = jnp.sum(masked, axis=1, keepdims=True)
        lq = jnp.log(masked / s_sum)
        ge = _gumbel_bits(((row * C + c) * P + col).astype(jnp.uint32), KE)
        e_idx = _first_argmax(lq + ge, col)
        e_alp = _gather_row(lq, col, e_idx)

        match = tour == e_idx
        pos = jnp.min(jnp.where(match, col, P), axis=1, keepdims=True)
        pos = jnp.where(pos == P, 0, pos)
        nxt = pos + 1
        nxt = jnp.where(nxt == P, 0, nxt)
        start_node = _gather_row(tour, col, pos)
        end_node = _gather_row(tour, col, nxt)

        btc_cols.append(b_idx)
        cc_cols.append(c_idx)
        alp_cols.append((b_alp + c_alp) + e_alp)
        e_cols.extend([start_node, end_node])

    btc_ref[...] = jnp.concatenate(btc_cols, axis=1)
    cc_ref[...] = jnp.concatenate(cc_cols, axis=1)
    etic_ref[...] = jnp.concatenate(e_cols, axis=1)
    alp_ref[...] = jnp.concatenate(alp_cols, axis=1)
    vc_ref[...] = jnp.broadcast_to(pv_ref[...], (ROWS, C))


@jax.jit
def _run(current_tour, predicted_value, backtrack_potentials,
         city_to_insert_probs, edge_to_insert_probs):
    grid = (B // ROWS,)
    row_spec = pl.BlockSpec((ROWS, P), lambda i: (i, 0))
    out_specs = [
        pl.BlockSpec((ROWS, C), lambda i: (i, 0)),
        pl.BlockSpec((ROWS, C), lambda i: (i, 0)),
        pl.BlockSpec((ROWS, 2 * C), lambda i: (i, 0)),
        pl.BlockSpec((ROWS, C), lambda i: (i, 0)),
        pl.BlockSpec((ROWS, C), lambda i: (i, 0)),
    ]
    out_shapes = [
        jax.ShapeDtypeStruct((B, C), jnp.int32),
        jax.ShapeDtypeStruct((B, C), jnp.int32),
        jax.ShapeDtypeStruct((B, 2 * C), jnp.int32),
        jax.ShapeDtypeStruct((B, C), jnp.float32),
        jax.ShapeDtypeStruct((B, C), jnp.float32),
    ]
    btc, cc, etic_flat, alp, vc = pl.pallas_call(
        _agent_kernel,
        grid=grid,
        in_specs=[
            row_spec,
            pl.BlockSpec((ROWS, 1), lambda i: (i, 0)),
            row_spec,
            row_spec,
            row_spec,
        ],
        out_specs=out_specs,
        out_shape=out_shapes,
    )(current_tour, predicted_value, backtrack_potentials,
      city_to_insert_probs, edge_to_insert_probs)
    return btc, cc, etic_flat.reshape(B, C, 2), alp, vc


def kernel(current_tour, predicted_value, backtrack_potentials,
           city_to_insert_probs, edge_to_insert_probs):
    return _run(current_tour, predicted_value, backtrack_potentials,
                city_to_insert_probs, edge_to_insert_probs)


# speedup vs baseline: 2.3691x; 1.2248x over previous
import jax
import jax.numpy as jnp
import numpy as np
from jax.experimental import pallas as pl
from jax.experimental.pallas import tpu as pltpu
from jax.sharding import Mesh, PartitionSpec as P_

B, P, C = 4096, 2000, 3

KB = (np.uint32(0x6D3E048F), np.uint32(0x1022172D))
KC = (np.uint32(0x03D7B32D), np.uint32(0xADD083F4))
KE = (np.uint32(0x92FB20EA), np.uint32(0x0F38D913))

TINY = np.float32(np.finfo(np.float32).tiny)

ROWS = 256


def _gumbel_bits(idx_u32, key):
    ks0 = jnp.uint32(key[0])
    ks1 = jnp.uint32(key[1])
    ks2 = ks0 ^ ks1 ^ jnp.uint32(0x1BD11BDA)
    ks = (ks0, ks1, ks2)
    x0 = jnp.full_like(idx_u32, ks0)
    x1 = idx_u32 + ks1
    rotations = ((13, 15, 26, 6), (17, 29, 16, 24))
    for i in range(5):
        for r in rotations[i % 2]:
            x0 = x0 + x1
            x1 = ((x1 << np.uint32(r)) | (x1 >> np.uint32(32 - r))) ^ x0
        x0 = x0 + ks[(i + 1) % 3]
        x1 = x1 + ks[(i + 2) % 3] + jnp.uint32(i + 1)
    bits = x0 ^ x1
    fb = (bits >> jnp.uint32(9)) | jnp.uint32(0x3F800000)
    f = jax.lax.bitcast_convert_type(fb, jnp.float32) - jnp.float32(1.0)
    u = jnp.maximum(f * jnp.float32(1.0) + TINY, TINY)
    return -jnp.log(-jnp.log(u))


def _first_argmax(s, col):
    m = jnp.max(s, axis=1, keepdims=True)
    return jnp.min(jnp.where(s == m, col, P), axis=1, keepdims=True)


def _gather_row(x, col, idx):
    return jnp.sum(jnp.where(col == idx, x, jnp.zeros_like(x)), axis=1,
                   keepdims=True)


def _agent_kernel(off_ref, tour_ref, pv_ref, bt_ref, cp_ref, ep_ref,
                  btc_ref, cc_ref, etic_ref, alp_ref, vc_ref):
    i = pl.program_id(0)
    bt = bt_ref[...]
    cp = cp_ref[...]
    ep = ep_ref[...]
    tour = tour_ref[...]

    row = (jax.lax.broadcasted_iota(jnp.int32, (ROWS, P), 0)
           + i * ROWS + off_ref[0])
    col = jax.lax.broadcasted_iota(jnp.int32, (ROWS, P), 1)

    bt_max = jnp.max(bt, axis=1, keepdims=True)
    bt_lse = jnp.log(jnp.sum(jnp.exp(bt - bt_max), axis=1, keepdims=True))

    cl = jnp.log(jnp.maximum(cp, jnp.float32(1e-30)))
    cp_max = jnp.max(cp, axis=1, keepdims=True)
    cp_lse = jnp.log(jnp.sum(jnp.exp(cp - cp_max), axis=1, keepdims=True))

    btc_cols, cc_cols, alp_cols, e_cols = [], [], [], []
    for c in range(C):
        gb = _gumbel_bits(((c * B + row) * P + col).astype(jnp.uint32), KB)
        b_idx = _first_argmax(bt + gb, col)
        b_alp = _gather_row(bt, col, b_idx) - bt_max - bt_lse

        gc = _gumbel_bits(((c * B + row) * P + col).astype(jnp.uint32), KC)
        c_idx = _first_argmax(cl + gc, col)
        c_alp = _gather_row(cp, col, c_idx) - cp_max - cp_lse

        masked = jnp.where(col == c_idx, jnp.float32(1e-9), ep)
        s_sum = jnp.sum(masked, axis=1, keepdims=True)
        lq = jnp.log(masked / s_sum)
        ge = _gumbel_bits(((row * C + c) * P + col).astype(jnp.uint32), KE)
        e_idx = _first_argmax(lq + ge, col)
        e_alp = _gather_row(lq, col, e_idx)

        match = tour == e_idx
        pos = jnp.min(jnp.where(match, col, P), axis=1, keepdims=True)
        pos = jnp.where(pos == P, 0, pos)
        nxt = pos + 1
        nxt = jnp.where(nxt == P, 0, nxt)
        start_node = _gather_row(tour, col, pos)
        end_node = _gather_row(tour, col, nxt)

        btc_cols.append(b_idx)
        cc_cols.append(c_idx)
        alp_cols.append((b_alp + c_alp) + e_alp)
        e_cols.extend([start_node, end_node])

    btc_ref[...] = jnp.concatenate(btc_cols, axis=1)
    cc_ref[...] = jnp.concatenate(cc_cols, axis=1)
    etic_ref[...] = jnp.concatenate(e_cols, axis=1)
    alp_ref[...] = jnp.concatenate(alp_cols, axis=1)
    vc_ref[...] = jnp.broadcast_to(pv_ref[...], (ROWS, C))


def _pallas_run(off, current_tour, predicted_value, backtrack_potentials,
                city_to_insert_probs, edge_to_insert_probs):
    rows_local = current_tour.shape[0]
    grid = (rows_local // ROWS,)
    row_spec = pl.BlockSpec((ROWS, P), lambda i: (i, 0))
    out_specs = [
        pl.BlockSpec((ROWS, C), lambda i: (i, 0)),
        pl.BlockSpec((ROWS, C), lambda i: (i, 0)),
        pl.BlockSpec((ROWS, 2 * C), lambda i: (i, 0)),
        pl.BlockSpec((ROWS, C), lambda i: (i, 0)),
        pl.BlockSpec((ROWS, C), lambda i: (i, 0)),
    ]
    out_shapes = [
        jax.ShapeDtypeStruct((rows_local, C), jnp.int32),
        jax.ShapeDtypeStruct((rows_local, C), jnp.int32),
        jax.ShapeDtypeStruct((rows_local, 2 * C), jnp.int32),
        jax.ShapeDtypeStruct((rows_local, C), jnp.float32),
        jax.ShapeDtypeStruct((rows_local, C), jnp.float32),
    ]
    return pl.pallas_call(
        _agent_kernel,
        grid=grid,
        in_specs=[
            pl.BlockSpec(memory_space=pltpu.SMEM),
            row_spec,
            pl.BlockSpec((ROWS, 1), lambda i: (i, 0)),
            row_spec,
            row_spec,
            row_spec,
        ],
        out_specs=out_specs,
        out_shape=out_shapes,
    )(off, current_tour, predicted_value, backtrack_potentials,
      city_to_insert_probs, edge_to_insert_probs)


def _num_shards():
    try:
        ndev = len(jax.devices())
    except RuntimeError:
        ndev = 1
    n = 1
    while n * 2 <= ndev and (B // (n * 2)) % ROWS == 0:
        n *= 2
    return n


_NSHARD = _num_shards()


@jax.jit
def _run(current_tour, predicted_value, backtrack_potentials,
         city_to_insert_probs, edge_to_insert_probs):
    if _NSHARD == 1:
        off = jnp.zeros((1,), jnp.int32)
        btc, cc, etic_flat, alp, vc = _pallas_run(
            off, current_tour, predicted_value, backtrack_potentials,
            city_to_insert_probs, edge_to_insert_probs)
        return btc, cc, etic_flat.reshape(B, C, 2), alp, vc

    mesh = Mesh(np.array(jax.devices()[:_NSHARD]), ("b",))
    shard_rows = B // _NSHARD

    def _body(tour, pv, bt, cp, ep):
        off = (jax.lax.axis_index("b") * shard_rows).astype(jnp.int32)
        return _pallas_run(off.reshape(1), tour, pv, bt, cp, ep)

    btc, cc, etic_flat, alp, vc = jax.shard_map(
        _body, mesh=mesh,
        in_specs=(P_("b"), P_("b"), P_("b"), P_("b"), P_("b")),
        out_specs=(P_("b"), P_("b"), P_("b"), P_("b"), P_("b")),
        check_vma=False,
    )(current_tour, predicted_value, backtrack_potentials,
      city_to_insert_probs, edge_to_insert_probs)
    return btc, cc, etic_flat.reshape(B, C, 2), alp, vc


def kernel(current_tour, predicted_value, backtrack_potentials,
           city_to_insert_probs, edge_to_insert_probs):
    return _run(current_tour, predicted_value, backtrack_potentials,
                city_to_insert_probs, edge_to_insert_probs)


# speedup vs baseline: 3.4396x; 1.4519x over previous
import jax
import jax.numpy as jnp
import numpy as np
from jax.experimental import pallas as pl
from jax.experimental.pallas import tpu as pltpu
from jax.sharding import Mesh, PartitionSpec as P_

B, P, C = 4096, 2000, 3

KB = (np.uint32(0x6D3E048F), np.uint32(0x1022172D))
KC = (np.uint32(0x03D7B32D), np.uint32(0xADD083F4))
KE = (np.uint32(0x92FB20EA), np.uint32(0x0F38D913))

TINY = np.float32(np.finfo(np.float32).tiny)

ROWS = 256


def _gumbel_bits(idx_u32, key):
    ks0 = jnp.uint32(key[0])
    ks1 = jnp.uint32(key[1])
    ks2 = ks0 ^ ks1 ^ jnp.uint32(0x1BD11BDA)
    ks = (ks0, ks1, ks2)
    x0 = jnp.full_like(idx_u32, ks0)
    x1 = idx_u32 + ks1
    rotations = ((13, 15, 26, 6), (17, 29, 16, 24))
    for i in range(5):
        for r in rotations[i % 2]:
            x0 = x0 + x1
            x1 = ((x1 << np.uint32(r)) | (x1 >> np.uint32(32 - r))) ^ x0
        x0 = x0 + ks[(i + 1) % 3]
        x1 = x1 + ks[(i + 2) % 3] + jnp.uint32(i + 1)
    bits = x0 ^ x1
    fb = (bits >> jnp.uint32(9)) | jnp.uint32(0x3F800000)
    f = jax.lax.bitcast_convert_type(fb, jnp.float32) - jnp.float32(1.0)
    u = jnp.maximum(f * jnp.float32(1.0) + TINY, TINY)
    return -jnp.log(-jnp.log(u))


def _first_argmax(s, col):
    m = jnp.max(s, axis=1, keepdims=True)
    return jnp.min(jnp.where(s == m, col, P), axis=1, keepdims=True)


def _gather_row(x, col, idx):
    return jnp.sum(jnp.where(col == idx, x, jnp.zeros_like(x)), axis=1,
                   keepdims=True)


def _agent_kernel(off_ref, tour_ref, pv_ref, bt_ref, cp_ref, ep_ref,
                  btc_ref, cc_ref, etic_ref, alp_ref, vc_ref):
    i = pl.program_id(0)
    bt = bt_ref[...]
    cp = cp_ref[...]
    ep = ep_ref[...]
    tour = tour_ref[...]

    row = (jax.lax.broadcasted_iota(jnp.int32, (ROWS, P), 0)
           + (off_ref[0] + i) * ROWS)
    col = jax.lax.broadcasted_iota(jnp.int32, (ROWS, P), 1)

    bt_max = jnp.max(bt, axis=1, keepdims=True)
    bt_lse = jnp.log(jnp.sum(jnp.exp(bt - bt_max), axis=1, keepdims=True))

    cl = jnp.log(jnp.maximum(cp, jnp.float32(1e-30)))
    cp_max = jnp.max(cp, axis=1, keepdims=True)
    cp_lse = jnp.log(jnp.sum(jnp.exp(cp - cp_max), axis=1, keepdims=True))

    btc_cols, cc_cols, alp_cols, e_cols = [], [], [], []
    for c in range(C):
        gb = _gumbel_bits(((c * B + row) * P + col).astype(jnp.uint32), KB)
        b_idx = _first_argmax(bt + gb, col)
        b_alp = _gather_row(bt, col, b_idx) - bt_max - bt_lse

        gc = _gumbel_bits(((c * B + row) * P + col).astype(jnp.uint32), KC)
        c_idx = _first_argmax(cl + gc, col)
        c_alp = _gather_row(cp, col, c_idx) - cp_max - cp_lse

        masked = jnp.where(col == c_idx, jnp.float32(1e-9), ep)
        s_sum = jnp.sum(masked, axis=1, keepdims=True)
        lq = jnp.log(masked / s_sum)
        ge = _gumbel_bits(((row * C + c) * P + col).astype(jnp.uint32), KE)
        e_idx = _first_argmax(lq + ge, col)
        e_alp = _gather_row(lq, col, e_idx)

        match = tour == e_idx
        pos = jnp.min(jnp.where(match, col, P), axis=1, keepdims=True)
        pos = jnp.where(pos == P, 0, pos)
        nxt = pos + 1
        nxt = jnp.where(nxt == P, 0, nxt)
        start_node = _gather_row(tour, col, pos)
        end_node = _gather_row(tour, col, nxt)

        btc_cols.append(b_idx)
        cc_cols.append(c_idx)
        alp_cols.append((b_alp + c_alp) + e_alp)
        e_cols.extend([start_node, end_node])

    btc_ref[...] = jnp.concatenate(btc_cols, axis=1)
    cc_ref[...] = jnp.concatenate(cc_cols, axis=1)
    etic_ref[...] = jnp.concatenate(e_cols, axis=1)
    alp_ref[...] = jnp.concatenate(alp_cols, axis=1)
    vc_ref[...] = jnp.broadcast_to(pv_ref[...], (ROWS, C))


def _pallas_run(off_blocks, rows_local, current_tour, predicted_value,
                backtrack_potentials, city_to_insert_probs,
                edge_to_insert_probs):
    grid = (rows_local // ROWS,)
    row_spec = pl.BlockSpec((ROWS, P), lambda i, off: (off[0] + i, 0))
    out_specs = [
        pl.BlockSpec((ROWS, C), lambda i, off: (i, 0)),
        pl.BlockSpec((ROWS, C), lambda i, off: (i, 0)),
        pl.BlockSpec((ROWS, 2 * C), lambda i, off: (i, 0)),
        pl.BlockSpec((ROWS, C), lambda i, off: (i, 0)),
        pl.BlockSpec((ROWS, C), lambda i, off: (i, 0)),
    ]
    out_shapes = [
        jax.ShapeDtypeStruct((rows_local, C), jnp.int32),
        jax.ShapeDtypeStruct((rows_local, C), jnp.int32),
        jax.ShapeDtypeStruct((rows_local, 2 * C), jnp.int32),
        jax.ShapeDtypeStruct((rows_local, C), jnp.float32),
        jax.ShapeDtypeStruct((rows_local, C), jnp.float32),
    ]
    grid_spec = pltpu.PrefetchScalarGridSpec(
        num_scalar_prefetch=1,
        grid=grid,
        in_specs=[
            row_spec,
            pl.BlockSpec((ROWS, 1), lambda i, off: (off[0] + i, 0)),
            row_spec,
            row_spec,
            row_spec,
        ],
        out_specs=out_specs,
    )
    return pl.pallas_call(
        _agent_kernel,
        grid_spec=grid_spec,
        out_shape=out_shapes,
    )(off_blocks, current_tour, predicted_value, backtrack_potentials,
      city_to_insert_probs, edge_to_insert_probs)


def _num_shards():
    try:
        ndev = len(jax.devices())
    except RuntimeError:
        ndev = 1
    n = 1
    while n * 2 <= ndev and (B // (n * 2)) % ROWS == 0:
        n *= 2
    return n


_NSHARD = _num_shards()


@jax.jit
def _run(current_tour, predicted_value, backtrack_potentials,
         city_to_insert_probs, edge_to_insert_probs):
    if _NSHARD == 1:
        off = jnp.zeros((1,), jnp.int32)
        btc, cc, etic_flat, alp, vc = _pallas_run(
            off, B, current_tour, predicted_value, backtrack_potentials,
            city_to_insert_probs, edge_to_insert_probs)
        return btc, cc, etic_flat.reshape(B, C, 2), alp, vc

    mesh = Mesh(np.array(jax.devices()[:_NSHARD]), ("b",))
    shard_rows = B // _NSHARD

    def _body(tour, pv, bt, cp, ep):
        off = (jax.lax.axis_index("b") * (shard_rows // ROWS)).astype(jnp.int32)
        return _pallas_run(off.reshape(1), shard_rows, tour, pv, bt, cp, ep)

    btc, cc, etic_flat, alp, vc = jax.shard_map(
        _body, mesh=mesh,
        in_specs=(P_(), P_(), P_(), P_(), P_()),
        out_specs=(P_("b"), P_("b"), P_("b"), P_("b"), P_("b")),
        check_vma=False,
    )(current_tour, predicted_value, backtrack_potentials,
      city_to_insert_probs, edge_to_insert_probs)
    return btc, cc, etic_flat.reshape(B, C, 2), alp, vc


def kernel(current_tour, predicted_value, backtrack_potentials,
           city_to_insert_probs, edge_to_insert_probs):
    return _run(current_tour, predicted_value, backtrack_potentials,
                city_to_insert_probs, edge_to_insert_probs)


# speedup vs baseline: 3.5456x; 1.0308x over previous
import jax
import jax.numpy as jnp
import numpy as np
from jax.experimental import pallas as pl
from jax.experimental.pallas import tpu as pltpu
from jax.sharding import Mesh, PartitionSpec as P_

B, P, C = 4096, 2000, 3

KB = (np.uint32(0x6D3E048F), np.uint32(0x1022172D))
KC = (np.uint32(0x03D7B32D), np.uint32(0xADD083F4))
KE = (np.uint32(0x92FB20EA), np.uint32(0x0F38D913))

TINY = np.float32(np.finfo(np.float32).tiny)

ROWS = 256


def _gumbel_bits(idx_u32, key):
    ks0 = jnp.uint32(key[0])
    ks1 = jnp.uint32(key[1])
    ks2 = ks0 ^ ks1 ^ jnp.uint32(0x1BD11BDA)
    ks = (ks0, ks1, ks2)
    x0 = jnp.full_like(idx_u32, ks0)
    x1 = idx_u32 + ks1
    rotations = ((13, 15, 26, 6), (17, 29, 16, 24))
    for i in range(5):
        for r in rotations[i % 2]:
            x0 = x0 + x1
            x1 = ((x1 << np.uint32(r)) | (x1 >> np.uint32(32 - r))) ^ x0
        x0 = x0 + ks[(i + 1) % 3]
        x1 = x1 + ks[(i + 2) % 3] + jnp.uint32(i + 1)
    bits = x0 ^ x1
    fb = (bits >> jnp.uint32(9)) | jnp.uint32(0x3F800000)
    f = jax.lax.bitcast_convert_type(fb, jnp.float32) - jnp.float32(1.0)
    u = jnp.maximum(f * jnp.float32(1.0) + TINY, TINY)
    return -jnp.log(-jnp.log(u))


def _first_argmax(s, col):
    m = jnp.max(s, axis=1, keepdims=True)
    eq = s == m
    idx = jnp.min(jnp.where(eq, col, P), axis=1, keepdims=True)
    return idx, eq


def _gather_row(x, col, idx):
    return jnp.sum(jnp.where(col == idx, x, jnp.zeros_like(x)), axis=1,
                   keepdims=True)


def _gather_mask(x, eq):
    return jnp.sum(jnp.where(eq, x, jnp.zeros_like(x)), axis=1, keepdims=True)


def _agent_kernel(off_ref, tour_ref, pv_ref, bt_ref, cp_ref, ep_ref,
                  btc_ref, cc_ref, etic_ref, alp_ref, vc_ref):
    i = pl.program_id(0)
    bt = bt_ref[...]
    cp = cp_ref[...]
    ep = ep_ref[...]
    tour = tour_ref[...]

    row = (jax.lax.broadcasted_iota(jnp.int32, (ROWS, P), 0)
           + (off_ref[0] + i) * ROWS)
    col = jax.lax.broadcasted_iota(jnp.int32, (ROWS, P), 1)

    bt_max = jnp.max(bt, axis=1, keepdims=True)
    bt_lse = jnp.log(jnp.sum(jnp.exp(bt - bt_max), axis=1, keepdims=True))

    cl = jnp.log(jnp.maximum(cp, jnp.float32(1e-30)))
    cp_max = jnp.max(cp, axis=1, keepdims=True)
    cp_lse = jnp.log(jnp.sum(jnp.exp(cp - cp_max), axis=1, keepdims=True))

    btc_cols, cc_cols, alp_cols, e_cols = [], [], [], []
    for c in range(C):
        gb = _gumbel_bits(((c * B + row) * P + col).astype(jnp.uint32), KB)
        b_idx, b_eq = _first_argmax(bt + gb, col)
        b_alp = _gather_mask(bt, b_eq) - bt_max - bt_lse

        gc = _gumbel_bits(((c * B + row) * P + col).astype(jnp.uint32), KC)
        c_idx, c_eq = _first_argmax(cl + gc, col)
        c_alp = _gather_mask(cp, c_eq) - cp_max - cp_lse

        masked = jnp.where(col == c_idx, jnp.float32(1e-9), ep)
        s_sum = jnp.sum(masked, axis=1, keepdims=True)
        lq = jnp.log(masked / s_sum)
        ge = _gumbel_bits(((row * C + c) * P + col).astype(jnp.uint32), KE)
        e_idx, e_eq = _first_argmax(lq + ge, col)
        e_alp = _gather_mask(lq, e_eq)

        match = tour == e_idx
        pos_raw = jnp.min(jnp.where(match, col, P), axis=1, keepdims=True)
        found = pos_raw < P
        pos = jnp.where(found, pos_raw, 0)
        nxt = pos + 1
        nxt = jnp.where(nxt == P, 0, nxt)
        start_node = jnp.where(found, e_idx, tour[:, 0:1])
        end_node = _gather_row(tour, col, nxt)

        btc_cols.append(b_idx)
        cc_cols.append(c_idx)
        alp_cols.append((b_alp + c_alp) + e_alp)
        e_cols.extend([start_node, end_node])

    btc_ref[...] = jnp.concatenate(btc_cols, axis=1)
    cc_ref[...] = jnp.concatenate(cc_cols, axis=1)
    etic_ref[...] = jnp.concatenate(e_cols, axis=1)
    alp_ref[...] = jnp.concatenate(alp_cols, axis=1)
    vc_ref[...] = jnp.broadcast_to(pv_ref[...], (ROWS, C))


def _pallas_run(off_blocks, rows_local, current_tour, predicted_value,
                backtrack_potentials, city_to_insert_probs,
                edge_to_insert_probs):
    grid = (rows_local // ROWS,)
    row_spec = pl.BlockSpec((ROWS, P), lambda i, off: (off[0] + i, 0))
    out_specs = [
        pl.BlockSpec((ROWS, C), lambda i, off: (i, 0)),
        pl.BlockSpec((ROWS, C), lambda i, off: (i, 0)),
        pl.BlockSpec((ROWS, 2 * C), lambda i, off: (i, 0)),
        pl.BlockSpec((ROWS, C), lambda i, off: (i, 0)),
        pl.BlockSpec((ROWS, C), lambda i, off: (i, 0)),
    ]
    out_shapes = [
        jax.ShapeDtypeStruct((rows_local, C), jnp.int32),
        jax.ShapeDtypeStruct((rows_local, C), jnp.int32),
        jax.ShapeDtypeStruct((rows_local, 2 * C), jnp.int32),
        jax.ShapeDtypeStruct((rows_local, C), jnp.float32),
        jax.ShapeDtypeStruct((rows_local, C), jnp.float32),
    ]
    grid_spec = pltpu.PrefetchScalarGridSpec(
        num_scalar_prefetch=1,
        grid=grid,
        in_specs=[
            row_spec,
            pl.BlockSpec((ROWS, 1), lambda i, off: (off[0] + i, 0)),
            row_spec,
            row_spec,
            row_spec,
        ],
        out_specs=out_specs,
    )
    return pl.pallas_call(
        _agent_kernel,
        grid_spec=grid_spec,
        out_shape=out_shapes,
    )(off_blocks, current_tour, predicted_value, backtrack_potentials,
      city_to_insert_probs, edge_to_insert_probs)


def _num_shards():
    try:
        ndev = len(jax.devices())
    except RuntimeError:
        ndev = 1
    n = 1
    while n * 2 <= ndev and (B // (n * 2)) % ROWS == 0:
        n *= 2
    return n


_NSHARD = _num_shards()


@jax.jit
def _run(current_tour, predicted_value, backtrack_potentials,
         city_to_insert_probs, edge_to_insert_probs):
    if _NSHARD == 1:
        off = jnp.zeros((1,), jnp.int32)
        btc, cc, etic_flat, alp, vc = _pallas_run(
            off, B, current_tour, predicted_value, backtrack_potentials,
            city_to_insert_probs, edge_to_insert_probs)
        return btc, cc, etic_flat.reshape(B, C, 2), alp, vc

    mesh = Mesh(np.array(jax.devices()[:_NSHARD]), ("b",))
    shard_rows = B // _NSHARD

    def _body(tour, pv, bt, cp, ep):
        off = (jax.lax.axis_index("b") * (shard_rows // ROWS)).astype(jnp.int32)
        return _pallas_run(off.reshape(1), shard_rows, tour, pv, bt, cp, ep)

    btc, cc, etic_flat, alp, vc = jax.shard_map(
        _body, mesh=mesh,
        in_specs=(P_(), P_(), P_(), P_(), P_()),
        out_specs=(P_("b"), P_("b"), P_("b"), P_("b"), P_("b")),
        check_vma=False,
    )(current_tour, predicted_value, backtrack_potentials,
      city_to_insert_probs, edge_to_insert_probs)
    return btc, cc, etic_flat.reshape(B, C, 2), alp, vc


def kernel(current_tour, predicted_value, backtrack_potentials,
           city_to_insert_probs, edge_to_insert_probs):
    return _run(current_tour, predicted_value, backtrack_potentials,
                city_to_insert_probs, edge_to_insert_probs)


# speedup vs baseline: 3.6386x; 1.0262x over previous
import jax
import jax.numpy as jnp
import numpy as np
from jax.experimental import pallas as pl
from jax.experimental.pallas import tpu as pltpu
from jax.sharding import Mesh, PartitionSpec as P_

B, P, C = 4096, 2000, 3

KB = (np.uint32(0x6D3E048F), np.uint32(0x1022172D))
KC = (np.uint32(0x03D7B32D), np.uint32(0xADD083F4))
KE = (np.uint32(0x92FB20EA), np.uint32(0x0F38D913))

TINY = np.float32(np.finfo(np.float32).tiny)

ROWS = 512


def _gumbel_bits(idx_u32, key):
    ks0 = jnp.uint32(key[0])
    ks1 = jnp.uint32(key[1])
    ks2 = ks0 ^ ks1 ^ jnp.uint32(0x1BD11BDA)
    ks = (ks0, ks1, ks2)
    x0 = jnp.full_like(idx_u32, ks0)
    x1 = idx_u32 + ks1
    rotations = ((13, 15, 26, 6), (17, 29, 16, 24))
    for i in range(5):
        for r in rotations[i % 2]:
            x0 = x0 + x1
            x1 = ((x1 << np.uint32(r)) | (x1 >> np.uint32(32 - r))) ^ x0
        x0 = x0 + ks[(i + 1) % 3]
        x1 = x1 + ks[(i + 2) % 3] + jnp.uint32(i + 1)
    bits = x0 ^ x1
    fb = (bits >> jnp.uint32(9)) | jnp.uint32(0x3F800000)
    f = jax.lax.bitcast_convert_type(fb, jnp.float32) - jnp.float32(1.0)
    u = jnp.maximum(f * jnp.float32(1.0) + TINY, TINY)
    return -jnp.log(-jnp.log(u))


def _first_argmax(s, col):
    m = jnp.max(s, axis=1, keepdims=True)
    eq = s == m
    idx = jnp.min(jnp.where(eq, col, P), axis=1, keepdims=True)
    return idx, eq


def _gather_row(x, col, idx):
    return jnp.sum(jnp.where(col == idx, x, jnp.zeros_like(x)), axis=1,
                   keepdims=True)


def _gather_mask(x, eq):
    return jnp.sum(jnp.where(eq, x, jnp.zeros_like(x)), axis=1, keepdims=True)


def _agent_kernel(off_ref, tour_ref, pv_ref, bt_ref, cp_ref, ep_ref,
                  btc_ref, cc_ref, etic_ref, alp_ref, vc_ref):
    i = pl.program_id(0)
    bt = bt_ref[...]
    cp = cp_ref[...]
    ep = ep_ref[...]
    tour = tour_ref[...]

    row = (jax.lax.broadcasted_iota(jnp.int32, (ROWS, P), 0)
           + (off_ref[0] + i) * ROWS)
    col = jax.lax.broadcasted_iota(jnp.int32, (ROWS, P), 1)

    bt_max = jnp.max(bt, axis=1, keepdims=True)
    bt_lse = jnp.log(jnp.sum(jnp.exp(bt - bt_max), axis=1, keepdims=True))

    cl = jnp.log(jnp.maximum(cp, jnp.float32(1e-30)))
    cp_max = jnp.max(cp, axis=1, keepdims=True)
    cp_lse = jnp.log(jnp.sum(jnp.exp(cp - cp_max), axis=1, keepdims=True))

    btc_cols, cc_cols, alp_cols, e_cols = [], [], [], []
    for c in range(C):
        gb = _gumbel_bits(((c * B + row) * P + col).astype(jnp.uint32), KB)
        b_idx, b_eq = _first_argmax(bt + gb, col)
        b_alp = _gather_mask(bt, b_eq) - bt_max - bt_lse

        gc = _gumbel_bits(((c * B + row) * P + col).astype(jnp.uint32), KC)
        c_idx, c_eq = _first_argmax(cl + gc, col)
        c_alp = _gather_mask(cp, c_eq) - cp_max - cp_lse

        masked = jnp.where(col == c_idx, jnp.float32(1e-9), ep)
        s_sum = jnp.sum(masked, axis=1, keepdims=True)
        lq = jnp.log(masked / s_sum)
        ge = _gumbel_bits(((row * C + c) * P + col).astype(jnp.uint32), KE)
        e_idx, e_eq = _first_argmax(lq + ge, col)
        e_alp = _gather_mask(lq, e_eq)

        match = tour == e_idx
        pos_raw = jnp.min(jnp.where(match, col, P), axis=1, keepdims=True)
        found = pos_raw < P
        pos = jnp.where(found, pos_raw, 0)
        nxt = pos + 1
        nxt = jnp.where(nxt == P, 0, nxt)
        start_node = jnp.where(found, e_idx, tour[:, 0:1])
        end_node = _gather_row(tour, col, nxt)

        btc_cols.append(b_idx)
        cc_cols.append(c_idx)
        alp_cols.append((b_alp + c_alp) + e_alp)
        e_cols.extend([start_node, end_node])

    btc_ref[...] = jnp.concatenate(btc_cols, axis=1)
    cc_ref[...] = jnp.concatenate(cc_cols, axis=1)
    etic_ref[...] = jnp.concatenate(e_cols, axis=1)
    alp_ref[...] = jnp.concatenate(alp_cols, axis=1)
    vc_ref[...] = jnp.broadcast_to(pv_ref[...], (ROWS, C))


def _pallas_run(off_blocks, rows_local, current_tour, predicted_value,
                backtrack_potentials, city_to_insert_probs,
                edge_to_insert_probs):
    grid = (rows_local // ROWS,)
    row_spec = pl.BlockSpec((ROWS, P), lambda i, off: (off[0] + i, 0))
    out_specs = [
        pl.BlockSpec((ROWS, C), lambda i, off: (i, 0)),
        pl.BlockSpec((ROWS, C), lambda i, off: (i, 0)),
        pl.BlockSpec((ROWS, 2 * C), lambda i, off: (i, 0)),
        pl.BlockSpec((ROWS, C), lambda i, off: (i, 0)),
        pl.BlockSpec((ROWS, C), lambda i, off: (i, 0)),
    ]
    out_shapes = [
        jax.ShapeDtypeStruct((rows_local, C), jnp.int32),
        jax.ShapeDtypeStruct((rows_local, C), jnp.int32),
        jax.ShapeDtypeStruct((rows_local, 2 * C), jnp.int32),
        jax.ShapeDtypeStruct((rows_local, C), jnp.float32),
        jax.ShapeDtypeStruct((rows_local, C), jnp.float32),
    ]
    grid_spec = pltpu.PrefetchScalarGridSpec(
        num_scalar_prefetch=1,
        grid=grid,
        in_specs=[
            row_spec,
            pl.BlockSpec((ROWS, 1), lambda i, off: (off[0] + i, 0)),
            row_spec,
            row_spec,
            row_spec,
        ],
        out_specs=out_specs,
    )
    return pl.pallas_call(
        _agent_kernel,
        grid_spec=grid_spec,
        out_shape=out_shapes,
    )(off_blocks, current_tour, predicted_value, backtrack_potentials,
      city_to_insert_probs, edge_to_insert_probs)


def _num_shards():
    try:
        ndev = len(jax.devices())
    except RuntimeError:
        ndev = 1
    n = 1
    while n * 2 <= ndev and (B // (n * 2)) % ROWS == 0:
        n *= 2
    return n


_NSHARD = _num_shards()


@jax.jit
def _run(current_tour, predicted_value, backtrack_potentials,
         city_to_insert_probs, edge_to_insert_probs):
    if _NSHARD == 1:
        off = jnp.zeros((1,), jnp.int32)
        btc, cc, etic_flat, alp, vc = _pallas_run(
            off, B, current_tour, predicted_value, backtrack_potentials,
            city_to_insert_probs, edge_to_insert_probs)
        return btc, cc, etic_flat.reshape(B, C, 2), alp, vc

    mesh = Mesh(np.array(jax.devices()[:_NSHARD]), ("b",))
    shard_rows = B // _NSHARD

    def _body(tour, pv, bt, cp, ep):
        off = (jax.lax.axis_index("b") * (shard_rows // ROWS)).astype(jnp.int32)
        return _pallas_run(off.reshape(1), shard_rows, tour, pv, bt, cp, ep)

    btc, cc, etic_flat, alp, vc = jax.shard_map(
        _body, mesh=mesh,
        in_specs=(P_(), P_(), P_(), P_(), P_()),
        out_specs=(P_("b"), P_("b"), P_("b"), P_("b"), P_("b")),
        check_vma=False,
    )(current_tour, predicted_value, backtrack_potentials,
      city_to_insert_probs, edge_to_insert_probs)
    return btc, cc, etic_flat.reshape(B, C, 2), alp, vc


def kernel(current_tour, predicted_value, backtrack_potentials,
           city_to_insert_probs, edge_to_insert_probs):
    return _run(current_tour, predicted_value, backtrack_potentials,
                city_to_insert_probs, edge_to_insert_probs)
